# Initial kernel scaffold; baseline (speedup 1.0000x reference)
import jax
import jax.numpy as jnp
from jax.experimental import pallas as pl


def kernel(x, sa0_val, fa0_val, sa1_val, fa1_val, hist0, hist1, W0, b0, W1, b1, g0, beta0, sample_ids_0, sample_ids_1, sample_ids_2, full_id_0, full_id_1, sa0_row, sa0_col, fa0_row, fa0_col, sa1_row, sa1_col, fa1_row, fa1_col):
    raise NotImplementedError("write your pallas kernel here")


# speedup vs baseline: 1.5945x; 1.5945x over previous
import functools

import jax
import jax.numpy as jnp
from jax import lax
from jax.experimental import pallas as pl
from jax.experimental.pallas import tpu as pltpu
from jax.experimental.pallas import tpu_sc as plsc

N = 100000
D = 128
N0, N1, N2 = 40000, 20000, 10000

NC = 2
NS = 16
NW = NC * NS
CH = 200
EW = 80
ZB = 200
QW = 32
NQC = 2

_F32 = jnp.float32


def _mesh():
    return plsc.VectorSubcoreMesh(core_axis_name="c", subcore_axis_name="s")


def _params():
    return pltpu.CompilerParams(use_tc_tiling_on_sc=False)


def _prep_body(x, h0, h1, s0, f0, s1, f1,
               hq, hf0, hs1, hf1,
               idxb, ga, gb, o0, o1, o2, o3, sem):
    c = lax.axis_index("c")
    s = lax.axis_index("s")
    wid = s * NC + c
    obufs = (o0, o1, o2, o3)

    def gather2(ids2d, cid, tbl, dst):
        pltpu.sync_copy(ids2d.at[pl.ds(cid * 8, 8)], idxb)
        for t in range(8):
            pltpu.async_copy(tbl.at[idxb.at[t]], dst.at[pl.ds(25 * t, 25)],
                             sem).wait()

    def split_rows(subtract):
        def body(e, carry):
            for q in range(4):
                for h in range(2):
                    sl = pl.ds(QW * q + 16 * h, 16)
                    a = ga[e, sl]
                    if subtract:
                        a = a - gb[e, sl]
                    obufs[q][e, pl.ds(16 * h, 16)] = a
            return carry
        lax.fori_loop(0, CH, body, 0)

    def run_job(nids, job):
        nch = nids // CH
        per = (nch + NW - 1) // NW

        def jbody(j, carry):
            cid = wid + NW * j

            @pl.when(cid < nch)
            def _():
                job(cid)
            return carry
        lax.fori_loop(0, per, jbody, 0)

    def a1(cid):
        gather2(s0, cid, x, ga)
        gather2(s0, cid, h0, gb)
        split_rows(True)
        for q in range(4):
            pltpu.sync_copy(obufs[q], hq.at[pl.ds(q * N0 + cid * CH, CH)])
    run_job(N0, a1)

    def a2(cid):
        gather2(f0, cid, h0, ga)
        split_rows(False)
        for q in range(4):
            pltpu.sync_copy(obufs[q], hf0.at[pl.ds(q * N0 + cid * CH, CH)])
    run_job(N0, a2)

    def a3(cid):
        gather2(s1, cid, h1, ga)
        pltpu.sync_copy(ga, hs1.at[pl.ds(cid * CH, CH)])
    run_job(N1, a3)

    def a4(cid):
        gather2(f1, cid, h1, ga)
        split_rows(False)
        for q in range(4):
            pltpu.sync_copy(obufs[q], hf1.at[pl.ds(q * N1 + cid * CH, CH)])
    run_job(N1, a4)


def _prep(x, hist0, hist1, s0, f0, s1, f1):
    out_type = (
        jax.ShapeDtypeStruct((4 * N0, QW), _F32),
        jax.ShapeDtypeStruct((4 * N0, QW), _F32),
        jax.ShapeDtypeStruct((N1, D), _F32),
        jax.ShapeDtypeStruct((4 * N1, QW), _F32),
    )
    scratch = [
        pltpu.VMEM((8, 25), jnp.int32),
        pltpu.VMEM((CH, D), _F32),
        pltpu.VMEM((CH, D), _F32),
        pltpu.VMEM((CH, QW), _F32),
        pltpu.VMEM((CH, QW), _F32),
        pltpu.VMEM((CH, QW), _F32),
        pltpu.VMEM((CH, QW), _F32),
        pltpu.SemaphoreType.DMA,
    ]
    return pl.kernel(_prep_body, out_type=out_type, mesh=_mesh(),
                     scratch_types=scratch, compiler_params=_params(),
                     )(x, hist0, hist1, s0, f0, s1, f1)


def _spmm_body(n_out, n_src,
               tA, cA, rA, vA, tB, cB, rB, vB,
               out, cidx, ridx, valb, rows, zbuf, acc, sem):
    c = lax.axis_index("c")
    s = lax.axis_index("s")
    nchz = n_out // ZB

    def zb(i, carry):
        for q in range(2):
            zbuf[i, pl.ds(16 * q, 16)] = jnp.zeros((16,), _F32)
        return carry
    lax.fori_loop(0, ZB, zb, 0)

    def run_edges(tbl, col, row, val, nch, coff):
        per = (nch + NS - 1) // NS

        def chunk(j, carry):
            cid = s + NS * j

            @pl.when(cid < nch)
            def _():
                b8 = cid * 8
                pltpu.sync_copy(col.at[pl.ds(b8, 8)], cidx)
                pltpu.sync_copy(row.at[pl.ds(b8, 8)], ridx)
                pltpu.sync_copy(val.at[pl.ds(b8, 8)], valb)
                for t in range(8):
                    for k16 in range(5):
                        sl = pl.ds(16 * k16, 16)
                        cidx[t, sl] = cidx[t, sl] + coff
                for t in range(8):
                    pltpu.async_copy(tbl.at[cidx.at[t]],
                                     rows.at[pl.ds(EW * t, EW)], sem).wait()

                def scale(g, carry2):
                    t8 = g // 5
                    g5 = g - t8 * 5
                    vv = valb[t8, pl.ds(16 * g5, 16)]
                    for l in range(16):
                        v = vv[l]
                        r = EW * t8 + 16 * g5 + l
                        for q in range(2):
                            sl = pl.ds(16 * q, 16)
                            rows[r, sl] = rows[r, sl] * v
                    return carry2
                lax.fori_loop(0, 40, scale, 0)
                for t in range(8):
                    pltpu.sync_copy(rows.at[pl.ds(EW * t, EW)],
                                    acc.at[ridx.at[t]], add=True)
            return carry
        lax.fori_loop(0, per, chunk, 0)

    for p in range(NQC):
        qidx = NQC * c + p

        def zero_chunk(j, carry):
            cid = s + NS * j

            @pl.when(cid < nchz)
            def _():
                pltpu.sync_copy(zbuf, acc.at[pl.ds(cid * ZB, ZB)])
            return carry
        lax.fori_loop(0, (nchz + NS - 1) // NS, zero_chunk, 0)
        plsc.subcore_barrier()

        coff = qidx * n_src
        run_edges(tA, cA, rA, vA, cA.shape[0] // 8, coff)
        run_edges(tB, cB, rB, vB, cB.shape[0] // 8, coff)
        plsc.subcore_barrier()

        def dump_chunk(j, carry):
            cid = s + NS * j

            @pl.when(cid < nchz)
            def _():
                pltpu.sync_copy(acc.at[pl.ds(cid * ZB, ZB)],
                                out.at[pl.ds(qidx * n_out + cid * ZB, ZB)])
            return carry
        lax.fori_loop(0, (nchz + NS - 1) // NS, dump_chunk, 0)
        plsc.subcore_barrier()


def _spmm(n_out, n_src, tA, cA, rA, vA, tB, cB, rB, vB):
    scratch = [
        pltpu.VMEM((8, EW), jnp.int32),
        pltpu.VMEM((8, EW), jnp.int32),
        pltpu.VMEM((8, EW), _F32),
        pltpu.VMEM((8 * EW, QW), _F32),
        pltpu.VMEM((ZB, QW), _F32),
        pltpu.VMEM_SHARED((n_out, QW), _F32),
        pltpu.SemaphoreType.DMA,
    ]
    body = functools.partial(_spmm_body, n_out, n_src)
    return pl.kernel(body, out_type=jax.ShapeDtypeStruct((4 * n_out, QW), _F32),
                     mesh=_mesh(), scratch_types=scratch,
                     compiler_params=_params(),
                     )(tA, cA, rA, vA, tB, cB, rB, vB)


_DN = (((1,), (1,)), ((), ()))


def _mm0(z0, w0, b0, g0, beta0, hs1):
    BR = 400
    nblk = N1 // BR

    def body(z0_ref, z1_ref, z2_ref, z3_ref, w_ref, b_ref, g_ref, bb_ref,
             hs_ref, o_ref):
        w = w_ref[...]
        z = lax.dot_general(z0_ref[...], w[:, 0 * QW:1 * QW], _DN,
                            preferred_element_type=_F32)
        z = z + lax.dot_general(z1_ref[...], w[:, 1 * QW:2 * QW], _DN,
                                preferred_element_type=_F32)
        z = z + lax.dot_general(z2_ref[...], w[:, 2 * QW:3 * QW], _DN,
                                preferred_element_type=_F32)
        z = z + lax.dot_general(z3_ref[...], w[:, 3 * QW:4 * QW], _DN,
                                preferred_element_type=_F32)
        z = z + b_ref[...]
        m = jnp.mean(z, axis=-1, keepdims=True)
        v = jnp.mean((z - m) ** 2, axis=-1, keepdims=True)
        z = (z - m) * lax.rsqrt(v + 1e-5) * g_ref[...] + bb_ref[...]
        z = jnp.maximum(z, 0.0) - hs_ref[...]
        for q in range(4):
            o_ref[q, 0] = z[:, QW * q:QW * (q + 1)]

    out = pl.pallas_call(
        body,
        grid=(nblk,),
        in_specs=[
            pl.BlockSpec((BR, QW), lambda i: (i, 0)),
            pl.BlockSpec((BR, QW), lambda i: (i + nblk, 0)),
            pl.BlockSpec((BR, QW), lambda i: (i + 2 * nblk, 0)),
            pl.BlockSpec((BR, QW), lambda i: (i + 3 * nblk, 0)),
            pl.BlockSpec((D, D), lambda i: (0, 0)),
            pl.BlockSpec((1, D), lambda i: (0, 0)),
            pl.BlockSpec((1, D), lambda i: (0, 0)),
            pl.BlockSpec((1, D), lambda i: (0, 0)),
            pl.BlockSpec((BR, D), lambda i: (i, 0)),
        ],
        out_specs=pl.BlockSpec((4, 1, BR, QW), lambda i: (0, i, 0, 0)),
        out_shape=jax.ShapeDtypeStruct((4, nblk, BR, QW), _F32),
    )(z0, z0, z0, z0, w0, b0.reshape(1, D), g0.reshape(1, D),
      beta0.reshape(1, D), hs1)
    return out.reshape(4 * N1, QW)


def _mm1(z2, w1, b1):
    BR = 400
    nblk = N2 // BR

    def body(z0_ref, z1_ref, z2_ref, z3_ref, w_ref, b_ref, o_ref):
        w = w_ref[...]
        z = lax.dot_general(z0_ref[...], w[:, 0 * QW:1 * QW], _DN,
                            preferred_element_type=_F32)
        z = z + lax.dot_general(z1_ref[...], w[:, 1 * QW:2 * QW], _DN,
                                preferred_element_type=_F32)
        z = z + lax.dot_general(z2_ref[...], w[:, 2 * QW:3 * QW], _DN,
                                preferred_element_type=_F32)
        z = z + lax.dot_general(z3_ref[...], w[:, 3 * QW:4 * QW], _DN,
                                preferred_element_type=_F32)
        z = z + b_ref[...]
        m = jnp.max(z, axis=-1, keepdims=True)
        zz = z - m
        lse = jnp.log(jnp.sum(jnp.exp(zz), axis=-1, keepdims=True))
        o_ref[...] = zz - lse

    return pl.pallas_call(
        body,
        grid=(nblk,),
        in_specs=[
            pl.BlockSpec((BR, QW), lambda i: (i, 0)),
            pl.BlockSpec((BR, QW), lambda i: (i + nblk, 0)),
            pl.BlockSpec((BR, QW), lambda i: (i + 2 * nblk, 0)),
            pl.BlockSpec((BR, QW), lambda i: (i + 3 * nblk, 0)),
            pl.BlockSpec((D, D), lambda i: (0, 0)),
            pl.BlockSpec((1, D), lambda i: (0, 0)),
        ],
        out_specs=pl.BlockSpec((BR, D), lambda i: (i, 0)),
        out_shape=jax.ShapeDtypeStruct((N2, D), _F32),
    )(z2, z2, z2, z2, w1, b1.reshape(1, D))


def kernel(x, sa0_val, fa0_val, sa1_val, fa1_val, hist0, hist1, W0, b0, W1, b1,
           g0, beta0, sample_ids_0, sample_ids_1, sample_ids_2, full_id_0,
           full_id_1, sa0_row, sa0_col, fa0_row, fa0_col, sa1_row, sa1_col,
           fa1_row, fa1_col):
    s0 = sample_ids_0.reshape(N0 // 25, 25)
    f0 = full_id_0.reshape(N0 // 25, 25)
    s1 = sample_ids_1.reshape(N1 // 25, 25)
    f1 = full_id_1.reshape(N1 // 25, 25)

    hq, hf0, hs1, hf1 = _prep(x, hist0, hist1, s0, f0, s1, f1)

    c0 = sa0_col.reshape(-1, EW)
    r0 = sa0_row.reshape(-1, EW)
    v0 = sa0_val.reshape(-1, EW)
    cf0 = fa0_col.reshape(-1, EW)
    rf0 = fa0_row.reshape(-1, EW)
    vf0 = fa0_val.reshape(-1, EW)
    z0 = _spmm(N1, N0, hq, c0, r0, v0, hf0, cf0, rf0, vf0)

    zin = _mm0(z0, W0, b0, g0, beta0, hs1)

    c1 = sa1_col.reshape(-1, EW)
    r1 = sa1_row.reshape(-1, EW)
    v1 = sa1_val.reshape(-1, EW)
    cf1 = fa1_col.reshape(-1, EW)
    rf1 = fa1_row.reshape(-1, EW)
    vf1 = fa1_val.reshape(-1, EW)
    z2 = _spmm(N2, N1, zin, c1, r1, v1, hf1, cf1, rf1, vf1)

    return _mm1(z2, W1, b1)


# speedup vs baseline: 3.7436x; 2.3479x over previous
import functools

import jax
import jax.numpy as jnp
from jax import lax
from jax.experimental import pallas as pl
from jax.experimental.pallas import tpu as pltpu
from jax.experimental.pallas import tpu_sc as plsc

N = 100000
D = 128
N0, N1, N2 = 40000, 20000, 10000

NC = 2
NS = 16
NW = NC * NS
CH = 200
EW = 80
ZB = 200
QW = 32
NQC = 2

_F32 = jnp.float32


def _mesh():
    return plsc.VectorSubcoreMesh(core_axis_name="c", subcore_axis_name="s")


def _params():
    return pltpu.CompilerParams(use_tc_tiling_on_sc=False)


def _prep_body(x, h0, h1, s0, f0, s1, f1,
               hq, hf0, hs1, hf1,
               idxb, ga, gb, o0, o1, o2, o3, sem):
    c = lax.axis_index("c")
    s = lax.axis_index("s")
    wid = s * NC + c
    obufs = (o0, o1, o2, o3)

    def gather2(ids2d, cid, tbl, dst):
        pltpu.sync_copy(ids2d.at[pl.ds(cid * 8, 8)], idxb)
        for t in range(8):
            pltpu.async_copy(tbl.at[idxb.at[t]], dst.at[pl.ds(25 * t, 25)],
                             sem).wait()

    def split_rows(subtract):
        def body(e, carry):
            for q in range(4):
                for h in range(2):
                    sl = pl.ds(QW * q + 16 * h, 16)
                    a = ga[e, sl]
                    if subtract:
                        a = a - gb[e, sl]
                    obufs[q][e, pl.ds(16 * h, 16)] = a
            return carry
        lax.fori_loop(0, CH, body, 0)

    def run_job(nids, job):
        nch = nids // CH
        per = (nch + NW - 1) // NW

        def jbody(j, carry):
            cid = wid + NW * j

            @pl.when(cid < nch)
            def _():
                job(cid)
            return carry
        lax.fori_loop(0, per, jbody, 0)

    def a1(cid):
        gather2(s0, cid, x, ga)
        gather2(s0, cid, h0, gb)
        split_rows(True)
        for q in range(4):
            pltpu.sync_copy(obufs[q], hq.at[pl.ds(q * N0 + cid * CH, CH)])
    run_job(N0, a1)

    def a2(cid):
        gather2(f0, cid, h0, ga)
        split_rows(False)
        for q in range(4):
            pltpu.sync_copy(obufs[q], hf0.at[pl.ds(q * N0 + cid * CH, CH)])
    run_job(N0, a2)

    def a3(cid):
        gather2(s1, cid, h1, ga)
        pltpu.sync_copy(ga, hs1.at[pl.ds(cid * CH, CH)])
    run_job(N1, a3)

    def a4(cid):
        gather2(f1, cid, h1, ga)
        split_rows(False)
        for q in range(4):
            pltpu.sync_copy(obufs[q], hf1.at[pl.ds(q * N1 + cid * CH, CH)])
    run_job(N1, a4)


def _prep(x, hist0, hist1, s0, f0, s1, f1):
    out_type = (
        jax.ShapeDtypeStruct((4 * N0, QW), _F32),
        jax.ShapeDtypeStruct((4 * N0, QW), _F32),
        jax.ShapeDtypeStruct((N1, D), _F32),
        jax.ShapeDtypeStruct((4 * N1, QW), _F32),
    )
    scratch = [
        pltpu.VMEM((8, 25), jnp.int32),
        pltpu.VMEM((CH, D), _F32),
        pltpu.VMEM((CH, D), _F32),
        pltpu.VMEM((CH, QW), _F32),
        pltpu.VMEM((CH, QW), _F32),
        pltpu.VMEM((CH, QW), _F32),
        pltpu.VMEM((CH, QW), _F32),
        pltpu.SemaphoreType.DMA,
    ]
    return pl.kernel(_prep_body, out_type=out_type, mesh=_mesh(),
                     scratch_types=scratch, compiler_params=_params(),
                     )(x, hist0, hist1, s0, f0, s1, f1)


def _spmm_body(n_out, n_src,
               tA, cA, rA, vA, tB, cB, rB, vB,
               out, cidx, ridx, valb, rows, zbuf, acc,
               semi, semg0, semg1, semsc):
    c = lax.axis_index("c")
    s = lax.axis_index("s")
    nchz = n_out // ZB
    semg = (semg0, semg1)

    def zb(i, carry):
        for q in range(2):
            zbuf[i, pl.ds(16 * q, 16)] = jnp.zeros((16,), _F32)
        return carry
    lax.fori_loop(0, ZB, zb, 0)

    def run_edges(tbl, col, row, val, nch, coff):
        per = (nch + NS - 1) // NS
        per2 = (per + 1) // 2

        def idx_load(jj, p):
            b8 = (s + NS * jj) * 8
            d1 = pltpu.async_copy(col.at[pl.ds(b8, 8)], cidx.at[p], semi)
            d2 = pltpu.async_copy(row.at[pl.ds(b8, 8)], ridx.at[p], semi)
            d3 = pltpu.async_copy(val.at[pl.ds(b8, 8)], valb.at[p], semi)
            d1.wait()
            d2.wait()
            d3.wait()
            for t in range(8):
                for k16 in range(5):
                    sl = pl.ds(16 * k16, 16)
                    cidx[p, t, sl] = cidx[p, t, sl] + coff

        def fire_gathers(p):
            for t in range(8):
                pltpu.async_copy(tbl.at[cidx.at[p].at[t]],
                                 rows.at[p].at[pl.ds(EW * t, EW)], semg[p])

        def drain_gathers(p):
            for t in range(8):
                pltpu.make_async_copy(tbl.at[cidx.at[p].at[t]],
                                     rows.at[p].at[pl.ds(EW * t, EW)],
                                     semg[p]).wait()

        def scale_scatter(p):
            for t in range(8):
                def scale(g5, carry2):
                    vv = valb[p, t, pl.ds(16 * g5, 16)]
                    for l in range(16):
                        v = vv[l]
                        r = EW * t + 16 * g5 + l
                        for q in range(2):
                            sl = pl.ds(16 * q, 16)
                            rows[p, r, sl] = rows[p, r, sl] * v
                    return carry2
                lax.fori_loop(0, 5, scale, 0)
                pltpu.async_copy(rows.at[p].at[pl.ds(EW * t, EW)],
                                 acc.at[ridx.at[p].at[t]], semsc, add=True)
            for t in range(8):
                pltpu.make_async_copy(rows.at[p].at[pl.ds(EW * t, EW)],
                                      acc.at[ridx.at[p].at[t]], semsc).wait()

        idx_load(0, 0)
        fire_gathers(0)

        def body2(j2, carry):
            for jj in range(2):
                j = 2 * j2 + jj
                p = jj
                cid = s + NS * j
                cidn = s + NS * (j + 1)

                @pl.when(cidn < nch)
                def _():
                    idx_load(j + 1, 1 - p)
                    fire_gathers(1 - p)

                @pl.when(cid < nch)
                def _():
                    drain_gathers(p)
                    scale_scatter(p)
            return carry
        lax.fori_loop(0, per2, body2, 0)

    for p in range(NQC):
        qidx = NQC * c + p

        def zero_chunk(j, carry):
            cid = s + NS * j

            @pl.when(cid < nchz)
            def _():
                pltpu.sync_copy(zbuf, acc.at[pl.ds(cid * ZB, ZB)])
            return carry
        lax.fori_loop(0, (nchz + NS - 1) // NS, zero_chunk, 0)
        plsc.subcore_barrier()

        coff = qidx * n_src
        run_edges(tA, cA, rA, vA, cA.shape[0] // 8, coff)
        run_edges(tB, cB, rB, vB, cB.shape[0] // 8, coff)
        plsc.subcore_barrier()

        def dump_chunk(j, carry):
            cid = s + NS * j

            @pl.when(cid < nchz)
            def _():
                pltpu.sync_copy(acc.at[pl.ds(cid * ZB, ZB)],
                                out.at[pl.ds(qidx * n_out + cid * ZB, ZB)])
            return carry
        lax.fori_loop(0, (nchz + NS - 1) // NS, dump_chunk, 0)
        plsc.subcore_barrier()


def _spmm(n_out, n_src, tA, cA, rA, vA, tB, cB, rB, vB):
    scratch = [
        pltpu.VMEM((2, 8, EW), jnp.int32),
        pltpu.VMEM((2, 8, EW), jnp.int32),
        pltpu.VMEM((2, 8, EW), _F32),
        pltpu.VMEM((2, 8 * EW, QW), _F32),
        pltpu.VMEM((ZB, QW), _F32),
        pltpu.VMEM_SHARED((n_out, QW), _F32),
        pltpu.SemaphoreType.DMA,
        pltpu.SemaphoreType.DMA,
        pltpu.SemaphoreType.DMA,
        pltpu.SemaphoreType.DMA,
    ]
    body = functools.partial(_spmm_body, n_out, n_src)
    return pl.kernel(body, out_type=jax.ShapeDtypeStruct((4 * n_out, QW), _F32),
                     mesh=_mesh(), scratch_types=scratch,
                     compiler_params=_params(),
                     )(tA, cA, rA, vA, tB, cB, rB, vB)


_DN = (((1,), (1,)), ((), ()))


def _mm0(z0, w0, b0, g0, beta0, hs1):
    BR = 400
    nblk = N1 // BR

    def body(z0_ref, z1_ref, z2_ref, z3_ref, w_ref, b_ref, g_ref, bb_ref,
             hs_ref, o_ref):
        w = w_ref[...]
        z = lax.dot_general(z0_ref[...], w[:, 0 * QW:1 * QW], _DN,
                            preferred_element_type=_F32)
        z = z + lax.dot_general(z1_ref[...], w[:, 1 * QW:2 * QW], _DN,
                                preferred_element_type=_F32)
        z = z + lax.dot_general(z2_ref[...], w[:, 2 * QW:3 * QW], _DN,
                                preferred_element_type=_F32)
        z = z + lax.dot_general(z3_ref[...], w[:, 3 * QW:4 * QW], _DN,
                                preferred_element_type=_F32)
        z = z + b_ref[...]
        m = jnp.mean(z, axis=-1, keepdims=True)
        v = jnp.mean((z - m) ** 2, axis=-1, keepdims=True)
        z = (z - m) * lax.rsqrt(v + 1e-5) * g_ref[...] + bb_ref[...]
        z = jnp.maximum(z, 0.0) - hs_ref[...]
        for q in range(4):
            o_ref[q, 0] = z[:, QW * q:QW * (q + 1)]

    out = pl.pallas_call(
        body,
        grid=(nblk,),
        in_specs=[
            pl.BlockSpec((BR, QW), lambda i: (i, 0)),
            pl.BlockSpec((BR, QW), lambda i: (i + nblk, 0)),
            pl.BlockSpec((BR, QW), lambda i: (i + 2 * nblk, 0)),
            pl.BlockSpec((BR, QW), lambda i: (i + 3 * nblk, 0)),
            pl.BlockSpec((D, D), lambda i: (0, 0)),
            pl.BlockSpec((1, D), lambda i: (0, 0)),
            pl.BlockSpec((1, D), lambda i: (0, 0)),
            pl.BlockSpec((1, D), lambda i: (0, 0)),
            pl.BlockSpec((BR, D), lambda i: (i, 0)),
        ],
        out_specs=pl.BlockSpec((4, 1, BR, QW), lambda i: (0, i, 0, 0)),
        out_shape=jax.ShapeDtypeStruct((4, nblk, BR, QW), _F32),
    )(z0, z0, z0, z0, w0, b0.reshape(1, D), g0.reshape(1, D),
      beta0.reshape(1, D), hs1)
    return out.reshape(4 * N1, QW)


def _mm1(z2, w1, b1):
    BR = 400
    nblk = N2 // BR

    def body(z0_ref, z1_ref, z2_ref, z3_ref, w_ref, b_ref, o_ref):
        w = w_ref[...]
        z = lax.dot_general(z0_ref[...], w[:, 0 * QW:1 * QW], _DN,
                            preferred_element_type=_F32)
        z = z + lax.dot_general(z1_ref[...], w[:, 1 * QW:2 * QW], _DN,
                                preferred_element_type=_F32)
        z = z + lax.dot_general(z2_ref[...], w[:, 2 * QW:3 * QW], _DN,
                                preferred_element_type=_F32)
        z = z + lax.dot_general(z3_ref[...], w[:, 3 * QW:4 * QW], _DN,
                                preferred_element_type=_F32)
        z = z + b_ref[...]
        m = jnp.max(z, axis=-1, keepdims=True)
        zz = z - m
        lse = jnp.log(jnp.sum(jnp.exp(zz), axis=-1, keepdims=True))
        o_ref[...] = zz - lse

    return pl.pallas_call(
        body,
        grid=(nblk,),
        in_specs=[
            pl.BlockSpec((BR, QW), lambda i: (i, 0)),
            pl.BlockSpec((BR, QW), lambda i: (i + nblk, 0)),
            pl.BlockSpec((BR, QW), lambda i: (i + 2 * nblk, 0)),
            pl.BlockSpec((BR, QW), lambda i: (i + 3 * nblk, 0)),
            pl.BlockSpec((D, D), lambda i: (0, 0)),
            pl.BlockSpec((1, D), lambda i: (0, 0)),
        ],
        out_specs=pl.BlockSpec((BR, D), lambda i: (i, 0)),
        out_shape=jax.ShapeDtypeStruct((N2, D), _F32),
    )(z2, z2, z2, z2, w1, b1.reshape(1, D))


def kernel(x, sa0_val, fa0_val, sa1_val, fa1_val, hist0, hist1, W0, b0, W1, b1,
           g0, beta0, sample_ids_0, sample_ids_1, sample_ids_2, full_id_0,
           full_id_1, sa0_row, sa0_col, fa0_row, fa0_col, sa1_row, sa1_col,
           fa1_row, fa1_col):
    s0 = sample_ids_0.reshape(N0 // 25, 25)
    f0 = full_id_0.reshape(N0 // 25, 25)
    s1 = sample_ids_1.reshape(N1 // 25, 25)
    f1 = full_id_1.reshape(N1 // 25, 25)

    hq, hf0, hs1, hf1 = _prep(x, hist0, hist1, s0, f0, s1, f1)

    c0 = sa0_col.reshape(-1, EW)
    r0 = sa0_row.reshape(-1, EW)
    v0 = sa0_val.reshape(-1, EW)
    cf0 = fa0_col.reshape(-1, EW)
    rf0 = fa0_row.reshape(-1, EW)
    vf0 = fa0_val.reshape(-1, EW)
    z0 = _spmm(N1, N0, hq, c0, r0, v0, hf0, cf0, rf0, vf0)

    zin = _mm0(z0, W0, b0, g0, beta0, hs1)

    c1 = sa1_col.reshape(-1, EW)
    r1 = sa1_row.reshape(-1, EW)
    v1 = sa1_val.reshape(-1, EW)
    cf1 = fa1_col.reshape(-1, EW)
    rf1 = fa1_row.reshape(-1, EW)
    vf1 = fa1_val.reshape(-1, EW)
    z2 = _spmm(N2, N1, zin, c1, r1, v1, hf1, cf1, rf1, vf1)

    return _mm1(z2, W1, b1)


# speedup vs baseline: 4.8497x; 1.2955x over previous
import functools

import jax
import jax.numpy as jnp
from jax import lax
from jax.experimental import pallas as pl
from jax.experimental.pallas import tpu as pltpu
from jax.experimental.pallas import tpu_sc as plsc

N = 100000
D = 128
N0, N1, N2 = 40000, 20000, 10000

NC = 2
NS = 16
NW = NC * NS
CH = 160
IW = 20
EW = 80
ZB = 200
QW = 32
NQC = 2

_F32 = jnp.float32


def _mesh():
    return plsc.VectorSubcoreMesh(core_axis_name="c", subcore_axis_name="s")


def _params():
    return pltpu.CompilerParams(use_tc_tiling_on_sc=False)


def _prep_body(x, h0, h1, s0, f0, s1, f1,
               hq, hf0, hs1, hf1,
               idxb, ga, gb, semi, semg0, semg1, semw):
    c = lax.axis_index("c")
    s = lax.axis_index("s")
    wid = s * NC + c
    semg = (semg0, semg1)

    def run_job(ids2d, nids, tbls, out, full_width):
        nch = nids // CH
        per = (nch + NW - 1) // NW
        per2 = (per + 1) // 2
        dsts = (ga, gb)

        def idx_load(jj, p):
            pltpu.async_copy(ids2d.at[pl.ds((wid + NW * jj) * 8, 8)],
                             idxb.at[p], semi).wait()

        def fire_gathers(p):
            for ti, tbl in enumerate(tbls):
                for t in range(8):
                    pltpu.async_copy(
                        tbl.at[idxb.at[p].at[t]],
                        dsts[ti].at[p].at[pl.ds(IW * t, IW)], semg[p])

        def drain_gathers(p):
            for ti, tbl in enumerate(tbls):
                for t in range(8):
                    pltpu.make_async_copy(
                        tbl.at[idxb.at[p].at[t]],
                        dsts[ti].at[p].at[pl.ds(IW * t, IW)], semg[p]).wait()

        def subtract(p):
            if len(tbls) == 2:
                def body(e, carry):
                    for q in range(8):
                        sl = pl.ds(16 * q, 16)
                        ga[p, e, sl] = ga[p, e, sl] - gb[p, e, sl]
                    return carry
                lax.fori_loop(0, CH, body, 0)

        def fire_writes(jj, p):
            cid = wid + NW * jj
            if full_width:
                pltpu.async_copy(ga.at[p], out.at[pl.ds(cid * CH, CH)], semw)
            else:
                for q in range(4):
                    pltpu.async_copy(
                        ga.at[p].at[:, pl.ds(QW * q, QW)],
                        out.at[pl.ds(q * nids + cid * CH, CH)], semw)

        def drain_writes(p):
            if full_width:
                pltpu.make_async_copy(ga.at[p], out.at[pl.ds(0, CH)],
                                      semw).wait()
            else:
                for q in range(4):
                    pltpu.make_async_copy(ga.at[p].at[:, pl.ds(QW * q, QW)],
                                          out.at[pl.ds(0, CH)], semw).wait()

        idx_load(0, 0)
        fire_gathers(0)

        def body2(j2, carry):
            for jj in range(2):
                j = 2 * j2 + jj
                p = jj
                cidn = wid + NW * (j + 1)
                cid = wid + NW * j
                drain_ok = (cidn < nch) if jj else ((cidn < nch) & (j2 >= 1))

                @pl.when(drain_ok)
                def _():
                    drain_writes(1 - p)

                @pl.when(cidn < nch)
                def _():
                    idx_load(j + 1, 1 - p)
                    fire_gathers(1 - p)

                @pl.when(cid < nch)
                def _():
                    drain_gathers(p)
                    subtract(p)
                    fire_writes(j, p)
            return carry
        lax.fori_loop(0, per2, body2, 0)
        drain_writes(0)
        drain_writes(1)

    run_job(s0, N0, (x, h0), hq, False)
    run_job(f0, N0, (h0,), hf0, False)
    run_job(s1, N1, (h1,), hs1, True)
    run_job(f1, N1, (h1,), hf1, False)


def _prep(x, hist0, hist1, s0, f0, s1, f1):
    out_type = (
        jax.ShapeDtypeStruct((4 * N0, QW), _F32),
        jax.ShapeDtypeStruct((4 * N0, QW), _F32),
        jax.ShapeDtypeStruct((N1, D), _F32),
        jax.ShapeDtypeStruct((4 * N1, QW), _F32),
    )
    scratch = [
        pltpu.VMEM((2, 8, IW), jnp.int32),
        pltpu.VMEM((2, CH, D), _F32),
        pltpu.VMEM((2, CH, D), _F32),
        pltpu.SemaphoreType.DMA,
        pltpu.SemaphoreType.DMA,
        pltpu.SemaphoreType.DMA,
        pltpu.SemaphoreType.DMA,
    ]
    return pl.kernel(_prep_body, out_type=out_type, mesh=_mesh(),
                     scratch_types=scratch, compiler_params=_params(),
                     )(x, hist0, hist1, s0, f0, s1, f1)


def _spmm_body(n_out, n_src,
               tA, cA, rA, vA, tB, cB, rB, vB,
               out, cidx, ridx, valb, rows, zbuf, acc,
               semi, semg0, semg1, semsc):
    c = lax.axis_index("c")
    s = lax.axis_index("s")
    nchz = n_out // ZB
    semg = (semg0, semg1)

    def zb(i, carry):
        for q in range(2):
            zbuf[i, pl.ds(16 * q, 16)] = jnp.zeros((16,), _F32)
        return carry
    lax.fori_loop(0, ZB, zb, 0)

    def run_edges(tbl, col, row, val, nch, coff):
        per = (nch + NS - 1) // NS
        per2 = (per + 1) // 2

        def idx_load(jj, p):
            b8 = (s + NS * jj) * 8
            d1 = pltpu.async_copy(col.at[pl.ds(b8, 8)], cidx.at[p], semi)
            d2 = pltpu.async_copy(row.at[pl.ds(b8, 8)], ridx.at[p], semi)
            d3 = pltpu.async_copy(val.at[pl.ds(b8, 8)], valb.at[p], semi)
            d1.wait()
            d2.wait()
            d3.wait()
            for t in range(8):
                for k16 in range(5):
                    sl = pl.ds(16 * k16, 16)
                    cidx[p, t, sl] = cidx[p, t, sl] + coff

        def fire_gathers(p):
            for t in range(8):
                pltpu.async_copy(tbl.at[cidx.at[p].at[t]],
                                 rows.at[p].at[pl.ds(EW * t, EW)], semg[p])

        def drain_gathers(p):
            for t in range(8):
                pltpu.make_async_copy(tbl.at[cidx.at[p].at[t]],
                                     rows.at[p].at[pl.ds(EW * t, EW)],
                                     semg[p]).wait()

        def scale_scatter(p):
            for t in range(8):
                def scale(g5, carry2):
                    vv = valb[p, t, pl.ds(16 * g5, 16)]
                    for l in range(16):
                        v = vv[l]
                        r = EW * t + 16 * g5 + l
                        for q in range(2):
                            sl = pl.ds(16 * q, 16)
                            rows[p, r, sl] = rows[p, r, sl] * v
                    return carry2
                lax.fori_loop(0, 5, scale, 0)
                pltpu.async_copy(rows.at[p].at[pl.ds(EW * t, EW)],
                                 acc.at[ridx.at[p].at[t]], semsc, add=True)
            for t in range(8):
                pltpu.make_async_copy(rows.at[p].at[pl.ds(EW * t, EW)],
                                      acc.at[ridx.at[p].at[t]], semsc).wait()

        idx_load(0, 0)
        fire_gathers(0)

        def body2(j2, carry):
            for jj in range(2):
                j = 2 * j2 + jj
                p = jj
                cid = s + NS * j
                cidn = s + NS * (j + 1)

                @pl.when(cidn < nch)
                def _():
                    idx_load(j + 1, 1 - p)
                    fire_gathers(1 - p)

                @pl.when(cid < nch)
                def _():
                    drain_gathers(p)
                    scale_scatter(p)
            return carry
        lax.fori_loop(0, per2, body2, 0)

    for p in range(NQC):
        qidx = NQC * c + p

        def zero_chunk(j, carry):
            cid = s + NS * j

            @pl.when(cid < nchz)
            def _():
                pltpu.sync_copy(zbuf, acc.at[pl.ds(cid * ZB, ZB)])
            return carry
        lax.fori_loop(0, (nchz + NS - 1) // NS, zero_chunk, 0)
        plsc.subcore_barrier()

        coff = qidx * n_src
        run_edges(tA, cA, rA, vA, cA.shape[0] // 8, coff)
        run_edges(tB, cB, rB, vB, cB.shape[0] // 8, coff)
        plsc.subcore_barrier()

        def dump_chunk(j, carry):
            cid = s + NS * j

            @pl.when(cid < nchz)
            def _():
                pltpu.sync_copy(acc.at[pl.ds(cid * ZB, ZB)],
                                out.at[pl.ds(qidx * n_out + cid * ZB, ZB)])
            return carry
        lax.fori_loop(0, (nchz + NS - 1) // NS, dump_chunk, 0)
        plsc.subcore_barrier()


def _spmm(n_out, n_src, tA, cA, rA, vA, tB, cB, rB, vB):
    scratch = [
        pltpu.VMEM((2, 8, EW), jnp.int32),
        pltpu.VMEM((2, 8, EW), jnp.int32),
        pltpu.VMEM((2, 8, EW), _F32),
        pltpu.VMEM((2, 8 * EW, QW), _F32),
        pltpu.VMEM((ZB, QW), _F32),
        pltpu.VMEM_SHARED((n_out, QW), _F32),
        pltpu.SemaphoreType.DMA,
        pltpu.SemaphoreType.DMA,
        pltpu.SemaphoreType.DMA,
        pltpu.SemaphoreType.DMA,
    ]
    body = functools.partial(_spmm_body, n_out, n_src)
    return pl.kernel(body, out_type=jax.ShapeDtypeStruct((4 * n_out, QW), _F32),
                     mesh=_mesh(), scratch_types=scratch,
                     compiler_params=_params(),
                     )(tA, cA, rA, vA, tB, cB, rB, vB)


_DN = (((1,), (1,)), ((), ()))


def _mm0(z0, w0, b0, g0, beta0, hs1):
    BR = 400
    nblk = N1 // BR

    def body(z0_ref, z1_ref, z2_ref, z3_ref, w_ref, b_ref, g_ref, bb_ref,
             hs_ref, o_ref):
        w = w_ref[...]
        z = lax.dot_general(z0_ref[...], w[:, 0 * QW:1 * QW], _DN,
                            preferred_element_type=_F32)
        z = z + lax.dot_general(z1_ref[...], w[:, 1 * QW:2 * QW], _DN,
                                preferred_element_type=_F32)
        z = z + lax.dot_general(z2_ref[...], w[:, 2 * QW:3 * QW], _DN,
                                preferred_element_type=_F32)
        z = z + lax.dot_general(z3_ref[...], w[:, 3 * QW:4 * QW], _DN,
                                preferred_element_type=_F32)
        z = z + b_ref[...]
        m = jnp.mean(z, axis=-1, keepdims=True)
        v = jnp.mean((z - m) ** 2, axis=-1, keepdims=True)
        z = (z - m) * lax.rsqrt(v + 1e-5) * g_ref[...] + bb_ref[...]
        z = jnp.maximum(z, 0.0) - hs_ref[...]
        for q in range(4):
            o_ref[q, 0] = z[:, QW * q:QW * (q + 1)]

    out = pl.pallas_call(
        body,
        grid=(nblk,),
        in_specs=[
            pl.BlockSpec((BR, QW), lambda i: (i, 0)),
            pl.BlockSpec((BR, QW), lambda i: (i + nblk, 0)),
            pl.BlockSpec((BR, QW), lambda i: (i + 2 * nblk, 0)),
            pl.BlockSpec((BR, QW), lambda i: (i + 3 * nblk, 0)),
            pl.BlockSpec((D, D), lambda i: (0, 0)),
            pl.BlockSpec((1, D), lambda i: (0, 0)),
            pl.BlockSpec((1, D), lambda i: (0, 0)),
            pl.BlockSpec((1, D), lambda i: (0, 0)),
            pl.BlockSpec((BR, D), lambda i: (i, 0)),
        ],
        out_specs=pl.BlockSpec((4, 1, BR, QW), lambda i: (0, i, 0, 0)),
        out_shape=jax.ShapeDtypeStruct((4, nblk, BR, QW), _F32),
    )(z0, z0, z0, z0, w0, b0.reshape(1, D), g0.reshape(1, D),
      beta0.reshape(1, D), hs1)
    return out.reshape(4 * N1, QW)


def _mm1(z2, w1, b1):
    BR = 400
    nblk = N2 // BR

    def body(z0_ref, z1_ref, z2_ref, z3_ref, w_ref, b_ref, o_ref):
        w = w_ref[...]
        z = lax.dot_general(z0_ref[...], w[:, 0 * QW:1 * QW], _DN,
                            preferred_element_type=_F32)
        z = z + lax.dot_general(z1_ref[...], w[:, 1 * QW:2 * QW], _DN,
                                preferred_element_type=_F32)
        z = z + lax.dot_general(z2_ref[...], w[:, 2 * QW:3 * QW], _DN,
                                preferred_element_type=_F32)
        z = z + lax.dot_general(z3_ref[...], w[:, 3 * QW:4 * QW], _DN,
                                preferred_element_type=_F32)
        z = z + b_ref[...]
        m = jnp.max(z, axis=-1, keepdims=True)
        zz = z - m
        lse = jnp.log(jnp.sum(jnp.exp(zz), axis=-1, keepdims=True))
        o_ref[...] = zz - lse

    return pl.pallas_call(
        body,
        grid=(nblk,),
        in_specs=[
            pl.BlockSpec((BR, QW), lambda i: (i, 0)),
            pl.BlockSpec((BR, QW), lambda i: (i + nblk, 0)),
            pl.BlockSpec((BR, QW), lambda i: (i + 2 * nblk, 0)),
            pl.BlockSpec((BR, QW), lambda i: (i + 3 * nblk, 0)),
            pl.BlockSpec((D, D), lambda i: (0, 0)),
            pl.BlockSpec((1, D), lambda i: (0, 0)),
        ],
        out_specs=pl.BlockSpec((BR, D), lambda i: (i, 0)),
        out_shape=jax.ShapeDtypeStruct((N2, D), _F32),
    )(z2, z2, z2, z2, w1, b1.reshape(1, D))


def kernel(x, sa0_val, fa0_val, sa1_val, fa1_val, hist0, hist1, W0, b0, W1, b1,
           g0, beta0, sample_ids_0, sample_ids_1, sample_ids_2, full_id_0,
           full_id_1, sa0_row, sa0_col, fa0_row, fa0_col, sa1_row, sa1_col,
           fa1_row, fa1_col):
    s0 = sample_ids_0.reshape(N0 // IW, IW)
    f0 = full_id_0.reshape(N0 // IW, IW)
    s1 = sample_ids_1.reshape(N1 // IW, IW)
    f1 = full_id_1.reshape(N1 // IW, IW)

    hq, hf0, hs1, hf1 = _prep(x, hist0, hist1, s0, f0, s1, f1)

    c0 = sa0_col.reshape(-1, EW)
    r0 = sa0_row.reshape(-1, EW)
    v0 = sa0_val.reshape(-1, EW)
    cf0 = fa0_col.reshape(-1, EW)
    rf0 = fa0_row.reshape(-1, EW)
    vf0 = fa0_val.reshape(-1, EW)
    z0 = _spmm(N1, N0, hq, c0, r0, v0, hf0, cf0, rf0, vf0)

    zin = _mm0(z0, W0, b0, g0, beta0, hs1)

    c1 = sa1_col.reshape(-1, EW)
    r1 = sa1_row.reshape(-1, EW)
    v1 = sa1_val.reshape(-1, EW)
    cf1 = fa1_col.reshape(-1, EW)
    rf1 = fa1_row.reshape(-1, EW)
    vf1 = fa1_val.reshape(-1, EW)
    z2 = _spmm(N2, N1, zin, c1, r1, v1, hf1, cf1, rf1, vf1)

    return _mm1(z2, W1, b1)


# speedup vs baseline: 4.8575x; 1.0016x over previous
import functools

import jax
import jax.numpy as jnp
from jax import lax
from jax.experimental import pallas as pl
from jax.experimental.pallas import tpu as pltpu
from jax.experimental.pallas import tpu_sc as plsc

N = 100000
D = 128
N0, N1, N2 = 40000, 20000, 10000

NC = 2
NS = 16
NW = NC * NS
CH = 160
IW = 20
EW = 80
CR = 8
ZB = 200
QW = 32
HW = 64
NQC = 2

_F32 = jnp.float32


def _mesh():
    return plsc.VectorSubcoreMesh(core_axis_name="c", subcore_axis_name="s")


def _params():
    return pltpu.CompilerParams(use_tc_tiling_on_sc=False)


def _prep_body(x, h0, h1, s0, f0, s1, f1,
               hq, hf0, hs1, hf1,
               idxb, ga, gb, semi, semg0, semg1, semw):
    c = lax.axis_index("c")
    s = lax.axis_index("s")
    wid = s * NC + c
    semg = (semg0, semg1)

    def run_job(ids2d, nids, tbls, out, nsplit):
        nch = nids // CH
        per = (nch + NW - 1) // NW
        per2 = (per + 1) // 2
        dsts = (ga, gb)

        def idx_load(jj, p):
            pltpu.async_copy(ids2d.at[pl.ds((wid + NW * jj) * 8, 8)],
                             idxb.at[p], semi).wait()

        def fire_gathers(p):
            for ti, tbl in enumerate(tbls):
                for t in range(8):
                    pltpu.async_copy(
                        tbl.at[idxb.at[p].at[t]],
                        dsts[ti].at[p].at[pl.ds(IW * t, IW)], semg[p])

        def drain_gathers(p):
            for ti, tbl in enumerate(tbls):
                for t in range(8):
                    pltpu.make_async_copy(
                        tbl.at[idxb.at[p].at[t]],
                        dsts[ti].at[p].at[pl.ds(IW * t, IW)], semg[p]).wait()

        def subtract(p):
            if len(tbls) == 2:
                def body(e, carry):
                    for q in range(8):
                        sl = pl.ds(16 * q, 16)
                        ga[p, e, sl] = ga[p, e, sl] - gb[p, e, sl]
                    return carry
                lax.fori_loop(0, CH, body, 0)

        sw = D // nsplit

        def fire_writes(jj, p):
            cid = wid + NW * jj
            if nsplit == 1:
                pltpu.async_copy(ga.at[p], out.at[pl.ds(cid * CH, CH)], semw)
            else:
                for q in range(nsplit):
                    pltpu.async_copy(
                        ga.at[p].at[:, pl.ds(sw * q, sw)],
                        out.at[pl.ds(q * nids + cid * CH, CH)], semw)

        def drain_writes(p):
            if nsplit == 1:
                pltpu.make_async_copy(ga.at[p], out.at[pl.ds(0, CH)],
                                      semw).wait()
            else:
                for q in range(nsplit):
                    pltpu.make_async_copy(ga.at[p].at[:, pl.ds(sw * q, sw)],
                                          out.at[pl.ds(0, CH)], semw).wait()

        idx_load(0, 0)
        fire_gathers(0)

        def body2(j2, carry):
            for jj in range(2):
                j = 2 * j2 + jj
                p = jj
                cidn = wid + NW * (j + 1)
                cid = wid + NW * j
                drain_ok = (cidn < nch) if jj else ((cidn < nch) & (j2 >= 1))

                @pl.when(drain_ok)
                def _():
                    drain_writes(1 - p)

                @pl.when(cidn < nch)
                def _():
                    idx_load(j + 1, 1 - p)
                    fire_gathers(1 - p)

                @pl.when(cid < nch)
                def _():
                    drain_gathers(p)
                    subtract(p)
                    fire_writes(j, p)
            return carry
        lax.fori_loop(0, per2, body2, 0)
        drain_writes(0)
        drain_writes(1)

    run_job(s0, N0, (x, h0), hq, 4)
    run_job(f0, N0, (h0,), hf0, 4)
    run_job(s1, N1, (h1,), hs1, 1)
    run_job(f1, N1, (h1,), hf1, 4)


def _prep(x, hist0, hist1, s0, f0, s1, f1):
    out_type = (
        jax.ShapeDtypeStruct((4 * N0, QW), _F32),
        jax.ShapeDtypeStruct((4 * N0, QW), _F32),
        jax.ShapeDtypeStruct((N1, D), _F32),
        jax.ShapeDtypeStruct((4 * N1, QW), _F32),
    )
    scratch = [
        pltpu.VMEM((2, 8, IW), jnp.int32),
        pltpu.VMEM((2, CH, D), _F32),
        pltpu.VMEM((2, CH, D), _F32),
        pltpu.SemaphoreType.DMA,
        pltpu.SemaphoreType.DMA,
        pltpu.SemaphoreType.DMA,
        pltpu.SemaphoreType.DMA,
    ]
    return pl.kernel(_prep_body, out_type=out_type, mesh=_mesh(),
                     scratch_types=scratch, compiler_params=_params(),
                     )(x, hist0, hist1, s0, f0, s1, f1)


def _spmm_body(n_out, n_src, qw, nqc,
               tA, cA, rA, vA, tB, cB, rB, vB,
               out, cidx, ridx, valb, rows, zbuf, acc,
               semi, semg0, semg1, semsc):
    c = lax.axis_index("c")
    s = lax.axis_index("s")
    nchz = n_out // ZB
    semg = (semg0, semg1)

    def zb(i, carry):
        for q in range(qw // 16):
            zbuf[i, pl.ds(16 * q, 16)] = jnp.zeros((16,), _F32)
        return carry
    lax.fori_loop(0, ZB, zb, 0)

    def run_edges(tbl, col, row, val, nch, coff):
        per = (nch + NS - 1) // NS
        per2 = (per + 1) // 2

        def idx_load(jj, p):
            b8 = (s + NS * jj) * CR
            d1 = pltpu.async_copy(col.at[pl.ds(b8, CR)], cidx.at[p], semi)
            d2 = pltpu.async_copy(row.at[pl.ds(b8, CR)], ridx.at[p], semi)
            d3 = pltpu.async_copy(val.at[pl.ds(b8, CR)], valb.at[p], semi)
            d1.wait()
            d2.wait()
            d3.wait()
            for t in range(CR):
                for k16 in range(5):
                    sl = pl.ds(16 * k16, 16)
                    cidx[p, t, sl] = cidx[p, t, sl] + coff

        def fire_gathers(p):
            for t in range(CR):
                pltpu.async_copy(tbl.at[cidx.at[p].at[t]],
                                 rows.at[p].at[pl.ds(EW * t, EW)], semg[p])

        def drain_gathers(p):
            for t in range(CR):
                pltpu.make_async_copy(tbl.at[cidx.at[p].at[t]],
                                     rows.at[p].at[pl.ds(EW * t, EW)],
                                     semg[p]).wait()

        def scale_scatter(p):
            for t in range(CR):
                def scale(g5, carry2):
                    vv = valb[p, t, pl.ds(16 * g5, 16)]
                    for l in range(16):
                        v = vv[l]
                        r = EW * t + 16 * g5 + l
                        for q in range(qw // 16):
                            sl = pl.ds(16 * q, 16)
                            rows[p, r, sl] = rows[p, r, sl] * v
                    return carry2
                lax.fori_loop(0, 5, scale, 0)
                pltpu.async_copy(rows.at[p].at[pl.ds(EW * t, EW)],
                                 acc.at[ridx.at[p].at[t]], semsc, add=True)
            for t in range(CR):
                pltpu.make_async_copy(rows.at[p].at[pl.ds(EW * t, EW)],
                                      acc.at[ridx.at[p].at[t]], semsc).wait()

        idx_load(0, 0)
        fire_gathers(0)

        def body2(j2, carry):
            for jj in range(2):
                j = 2 * j2 + jj
                p = jj
                cid = s + NS * j
                cidn = s + NS * (j + 1)

                @pl.when(cidn < nch)
                def _():
                    idx_load(j + 1, 1 - p)
                    fire_gathers(1 - p)

                @pl.when(cid < nch)
                def _():
                    drain_gathers(p)
                    scale_scatter(p)
            return carry
        lax.fori_loop(0, per2, body2, 0)

    for p in range(nqc):
        qidx = nqc * c + p

        def zero_chunk(j, carry):
            cid = s + NS * j

            @pl.when(cid < nchz)
            def _():
                pltpu.sync_copy(zbuf, acc.at[pl.ds(cid * ZB, ZB)])
            return carry
        lax.fori_loop(0, (nchz + NS - 1) // NS, zero_chunk, 0)
        plsc.subcore_barrier()

        coff = qidx * n_src
        run_edges(tA, cA, rA, vA, cA.shape[0] // CR, coff)
        run_edges(tB, cB, rB, vB, cB.shape[0] // CR, coff)
        plsc.subcore_barrier()

        def dump_chunk(j, carry):
            cid = s + NS * j

            @pl.when(cid < nchz)
            def _():
                pltpu.sync_copy(acc.at[pl.ds(cid * ZB, ZB)],
                                out.at[pl.ds(qidx * n_out + cid * ZB, ZB)])
            return carry
        lax.fori_loop(0, (nchz + NS - 1) // NS, dump_chunk, 0)
        plsc.subcore_barrier()


def _spmm(n_out, n_src, qw, nqc, tA, cA, rA, vA, tB, cB, rB, vB):
    scratch = [
        pltpu.VMEM((2, CR, EW), jnp.int32),
        pltpu.VMEM((2, CR, EW), jnp.int32),
        pltpu.VMEM((2, CR, EW), _F32),
        pltpu.VMEM((2, CR * EW, qw), _F32),
        pltpu.VMEM((ZB, qw), _F32),
        pltpu.VMEM_SHARED((n_out, qw), _F32),
        pltpu.SemaphoreType.DMA,
        pltpu.SemaphoreType.DMA,
        pltpu.SemaphoreType.DMA,
        pltpu.SemaphoreType.DMA,
    ]
    body = functools.partial(_spmm_body, n_out, n_src, qw, nqc)
    return pl.kernel(body,
                     out_type=jax.ShapeDtypeStruct((NC * nqc * n_out, qw),
                                                   _F32),
                     mesh=_mesh(), scratch_types=scratch,
                     compiler_params=_params(),
                     )(tA, cA, rA, vA, tB, cB, rB, vB)


_DN = (((1,), (1,)), ((), ()))


def _mm0(z0, w0, b0, g0, beta0, hs1):
    BR = 400
    nblk = N1 // BR

    def body(z0_ref, z1_ref, z2_ref, z3_ref, w_ref, b_ref, g_ref, bb_ref,
             hs_ref, o_ref):
        w = w_ref[...]
        z = lax.dot_general(z0_ref[...], w[:, 0 * QW:1 * QW], _DN,
                            preferred_element_type=_F32)
        z = z + lax.dot_general(z1_ref[...], w[:, 1 * QW:2 * QW], _DN,
                                preferred_element_type=_F32)
        z = z + lax.dot_general(z2_ref[...], w[:, 2 * QW:3 * QW], _DN,
                                preferred_element_type=_F32)
        z = z + lax.dot_general(z3_ref[...], w[:, 3 * QW:4 * QW], _DN,
                                preferred_element_type=_F32)
        z = z + b_ref[...]
        m = jnp.mean(z, axis=-1, keepdims=True)
        v = jnp.mean((z - m) ** 2, axis=-1, keepdims=True)
        z = (z - m) * lax.rsqrt(v + 1e-5) * g_ref[...] + bb_ref[...]
        z = jnp.maximum(z, 0.0) - hs_ref[...]
        for q in range(4):
            o_ref[q, 0] = z[:, QW * q:QW * (q + 1)]

    out = pl.pallas_call(
        body,
        grid=(nblk,),
        in_specs=[
            pl.BlockSpec((BR, QW), lambda i: (i, 0)),
            pl.BlockSpec((BR, QW), lambda i: (i + nblk, 0)),
            pl.BlockSpec((BR, QW), lambda i: (i + 2 * nblk, 0)),
            pl.BlockSpec((BR, QW), lambda i: (i + 3 * nblk, 0)),
            pl.BlockSpec((D, D), lambda i: (0, 0)),
            pl.BlockSpec((1, D), lambda i: (0, 0)),
            pl.BlockSpec((1, D), lambda i: (0, 0)),
            pl.BlockSpec((1, D), lambda i: (0, 0)),
            pl.BlockSpec((BR, D), lambda i: (i, 0)),
        ],
        out_specs=pl.BlockSpec((4, 1, BR, QW), lambda i: (0, i, 0, 0)),
        out_shape=jax.ShapeDtypeStruct((4, nblk, BR, QW), _F32),
    )(z0, z0, z0, z0, w0, b0.reshape(1, D), g0.reshape(1, D),
      beta0.reshape(1, D), hs1)
    return out.reshape(4 * N1, QW)


def _mm1(z2, w1, b1):
    BR = 400
    nblk = N2 // BR

    def body(z0_ref, z1_ref, z2_ref, z3_ref, w_ref, b_ref, o_ref):
        w = w_ref[...]
        z = lax.dot_general(z0_ref[...], w[:, 0 * QW:1 * QW], _DN,
                            preferred_element_type=_F32)
        z = z + lax.dot_general(z1_ref[...], w[:, 1 * QW:2 * QW], _DN,
                                preferred_element_type=_F32)
        z = z + lax.dot_general(z2_ref[...], w[:, 2 * QW:3 * QW], _DN,
                                preferred_element_type=_F32)
        z = z + lax.dot_general(z3_ref[...], w[:, 3 * QW:4 * QW], _DN,
                                preferred_element_type=_F32)
        z = z + b_ref[...]
        m = jnp.max(z, axis=-1, keepdims=True)
        zz = z - m
        lse = jnp.log(jnp.sum(jnp.exp(zz), axis=-1, keepdims=True))
        o_ref[...] = zz - lse

    return pl.pallas_call(
        body,
        grid=(nblk,),
        in_specs=[
            pl.BlockSpec((BR, QW), lambda i: (i, 0)),
            pl.BlockSpec((BR, QW), lambda i: (i + nblk, 0)),
            pl.BlockSpec((BR, QW), lambda i: (i + 2 * nblk, 0)),
            pl.BlockSpec((BR, QW), lambda i: (i + 3 * nblk, 0)),
            pl.BlockSpec((D, D), lambda i: (0, 0)),
            pl.BlockSpec((1, D), lambda i: (0, 0)),
        ],
        out_specs=pl.BlockSpec((BR, D), lambda i: (i, 0)),
        out_shape=jax.ShapeDtypeStruct((N2, D), _F32),
    )(z2, z2, z2, z2, w1, b1.reshape(1, D))


def kernel(x, sa0_val, fa0_val, sa1_val, fa1_val, hist0, hist1, W0, b0, W1, b1,
           g0, beta0, sample_ids_0, sample_ids_1, sample_ids_2, full_id_0,
           full_id_1, sa0_row, sa0_col, fa0_row, fa0_col, sa1_row, sa1_col,
           fa1_row, fa1_col):
    s0 = sample_ids_0.reshape(N0 // IW, IW)
    f0 = full_id_0.reshape(N0 // IW, IW)
    s1 = sample_ids_1.reshape(N1 // IW, IW)
    f1 = full_id_1.reshape(N1 // IW, IW)

    hq, hf0, hs1, hf1 = _prep(x, hist0, hist1, s0, f0, s1, f1)

    c0 = sa0_col.reshape(-1, EW)
    r0 = sa0_row.reshape(-1, EW)
    v0 = sa0_val.reshape(-1, EW)
    cf0 = fa0_col.reshape(-1, EW)
    rf0 = fa0_row.reshape(-1, EW)
    vf0 = fa0_val.reshape(-1, EW)
    z0 = _spmm(N1, N0, QW, NQC, hq, c0, r0, v0, hf0, cf0, rf0, vf0)

    zin = _mm0(z0, W0, b0, g0, beta0, hs1)

    c1 = sa1_col.reshape(-1, EW)
    r1 = sa1_row.reshape(-1, EW)
    v1 = sa1_val.reshape(-1, EW)
    cf1 = fa1_col.reshape(-1, EW)
    rf1 = fa1_row.reshape(-1, EW)
    vf1 = fa1_val.reshape(-1, EW)
    z2 = _spmm(N2, N1, QW, NQC, zin, c1, r1, v1, hf1, cf1, rf1, vf1)

    return _mm1(z2, W1, b1)
